# baseline (speedup 1.0000x reference)
import functools

import jax
import jax.numpy as jnp
from jax import lax
from jax.experimental import pallas as pl
from jax.experimental.pallas import tpu as pltpu
from jax.experimental.pallas import tpu_sc as plsc

_INFO = plsc.get_sparse_core_info()
_NC, _NS = _INFO.num_cores, _INFO.num_subcores
_NW = _NC * _NS
_L = 16

_CHUNK = 256
_PAIRS = _CHUNK // 2
_NBUF = 4
_DO_GATHER = True
_DO_WRITE = True


@functools.partial(jax.jit, static_argnames=("n", "d"))
def _emb_lookup(tp, idx1d, dummy, *, n, d):
    per_w = n // _NW
    n_chunks = per_w // _CHUNK
    assert (n_chunks - 2 * _NBUF) % _NBUF == 0 and n_chunks > 3 * _NBUF
    mesh = plsc.VectorSubcoreMesh(core_axis_name="c", subcore_axis_name="s")

    @functools.partial(
        pl.kernel,
        mesh=mesh,
        out_type=jax.ShapeDtypeStruct((n // 2, 2 * d), jnp.float32),
        scratch_types=[
            pltpu.VMEM((per_w,), jnp.int32),
            pltpu.VMEM((_NBUF, _PAIRS), jnp.int32),
            pltpu.VMEM((_NBUF, _PAIRS, 2 * d), jnp.float32),
            pltpu.SemaphoreType.DMA,
            pltpu.SemaphoreType.DMA,
            pltpu.SemaphoreType.DMA,
            pltpu.SemaphoreType.DMA,
            pltpu.SemaphoreType.DMA,
            pltpu.SemaphoreType.DMA,
            pltpu.SemaphoreType.DMA,
            pltpu.SemaphoreType.DMA,
        ],
    )
    def k(tp_hbm, idx_hbm, dummy_hbm, out_hbm, idx_all, pid_v, pairs_f,
          *sems):
        sem_g = sems[:_NBUF]
        sem_w = sems[_NBUF:]
        wid = lax.axis_index("s") * _NC + lax.axis_index("c")
        idx0 = wid * per_w
        pair0 = idx0 // 2
        lane = lax.iota(jnp.int32, _L)
        low_half = lane < 8
        ev_sel = jnp.arange(0, 2 * _L, 2, dtype=jnp.int32) % _L
        od_sel = ev_sel + 1

        def deinterleave(vv, sel):
            return vv.at[sel].get(mode="promise_in_bounds")

        def fire_gather(j, b):
            base = j * _CHUNK
            for g in range(_PAIRS // _L):
                v0 = idx_all[pl.ds(base + 2 * _L * g, _L)]
                v1 = idx_all[pl.ds(base + 2 * _L * g + _L, _L)]
                ev = jnp.where(low_half, deinterleave(v0, ev_sel),
                               deinterleave(v1, ev_sel))
                od = jnp.where(low_half, deinterleave(v0, od_sel),
                               deinterleave(v1, od_sel))
                pid_v[b, pl.ds(g * _L, _L)] = ev * 8 + od
            if _DO_GATHER:
                pltpu.make_async_copy(
                    tp_hbm.at[wid].at[pid_v.at[b]], pairs_f.at[b],
                    sem_g[b]).start()

        def wait_gather(b):
            if _DO_GATHER:
                pltpu.make_async_copy(dummy_hbm, pairs_f.at[b],
                                      sem_g[b]).wait()

        def fire_write(j, b):
            pb = pl.multiple_of(pair0 + j * _PAIRS, _PAIRS)
            if _DO_WRITE:
                pltpu.make_async_copy(
                    pairs_f.at[b], out_hbm.at[pl.ds(pb, _PAIRS)],
                    sem_w[b]).start()

        def drain_write(b):
            if _DO_WRITE:
                pltpu.make_async_copy(
                    pairs_f.at[b], out_hbm.at[pl.ds(0, _PAIRS)],
                    sem_w[b]).wait()

        ib = pl.multiple_of(idx0, _CHUNK)
        pltpu.sync_copy(idx_hbm.at[pl.ds(ib, per_w)], idx_all)

        def step(j, b, drain, fire):
            nxt = (b + 2) % _NBUF
            if drain:
                drain_write(nxt)
            if fire:
                fire_gather(j + 2, nxt)
            wait_gather(b)
            fire_write(j, b)

        fire_gather(0, 0)
        fire_gather(1, 1)
        for j in range(_NBUF):
            step(j, j, drain=(j >= 2), fire=True)

        def body(k_, carry):
            for b in range(_NBUF):
                step(k_ * _NBUF + b, b, drain=True, fire=True)
            return carry

        lax.fori_loop(1, n_chunks // _NBUF - 1, body, 0)

        for j in range(n_chunks - _NBUF, n_chunks):
            fire = j + 2 < n_chunks
            step(j, j % _NBUF, drain=fire, fire=fire)
        for b in range(_NBUF):
            drain_write(b)

    return k(tp, idx1d, dummy)


def kernel(actions, table):
    B, T, Hp, Wp = actions.shape
    n = B * T * Hp * Wp
    v, d = table.shape
    idx1d = actions.reshape(n).astype(jnp.int32)
    tp = jnp.concatenate(
        [jnp.repeat(table, v, axis=0), jnp.tile(table, (v, 1))], axis=1)
    tp = jnp.broadcast_to(tp, (_NW,) + tp.shape)
    dummy = jnp.zeros((_PAIRS, 2 * d), jnp.float32)
    out2 = _emb_lookup(tp, idx1d, dummy, n=n, d=d)
    return out2.reshape(B, T, Hp, Wp, d)


# speedup vs baseline: 1.4001x; 1.4001x over previous
import functools

import jax
import jax.numpy as jnp
from jax import lax
from jax.experimental import pallas as pl
from jax.experimental.pallas import tpu as pltpu
from jax.experimental.pallas import tpu_sc as plsc

_INFO = plsc.get_sparse_core_info()
_NC, _NS = _INFO.num_cores, _INFO.num_subcores
_NW = _NC * _NS
_L = 16

_CHUNK = 256
_PAIRS = _CHUNK // 2
_NBUF = 4
_DO_GATHER = True
_DO_WRITE = True


@functools.partial(jax.jit, static_argnames=("n", "d"))
def _emb_lookup(tp, idx1d, dummy, *, n, d):
    per_w = n // _NW
    n_chunks = per_w // _CHUNK
    assert (n_chunks - 2 * _NBUF) % _NBUF == 0 and n_chunks > 3 * _NBUF
    mesh = plsc.VectorSubcoreMesh(core_axis_name="c", subcore_axis_name="s")

    @functools.partial(
        pl.kernel,
        mesh=mesh,
        out_type=jax.ShapeDtypeStruct((n // 2, 2 * d), jnp.float32),
        scratch_types=[
            pltpu.VMEM((per_w,), jnp.int32),
            pltpu.VMEM((_NBUF, _PAIRS), jnp.int32),
            pltpu.VMEM((_NBUF, _PAIRS, 2 * d), jnp.float32),
            pltpu.SemaphoreType.DMA,
            pltpu.SemaphoreType.DMA,
            pltpu.SemaphoreType.DMA,
            pltpu.SemaphoreType.DMA,
            pltpu.SemaphoreType.DMA,
            pltpu.SemaphoreType.DMA,
            pltpu.SemaphoreType.DMA,
            pltpu.SemaphoreType.DMA,
        ],
    )
    def k(tp_hbm, idx_hbm, dummy_hbm, out_hbm, idx_all, pid_v, pairs_f,
          *sems):
        sem_g = sems[:_NBUF]
        sem_w = sems[_NBUF:]
        wid = lax.axis_index("s") * _NC + lax.axis_index("c")
        idx0 = wid * per_w
        pair0 = idx0 // 2
        lane = lax.iota(jnp.int32, _L)
        low_half = lane < 8
        ev_sel = jnp.arange(0, 2 * _L, 2, dtype=jnp.int32) % _L
        od_sel = ev_sel + 1

        def deinterleave(vv, sel):
            return vv.at[sel].get(mode="promise_in_bounds")

        def fire_gather(j, b):
            base = j * _CHUNK
            for g in range(_PAIRS // _L):
                v0 = idx_all[pl.ds(base + 2 * _L * g, _L)]
                v1 = idx_all[pl.ds(base + 2 * _L * g + _L, _L)]
                ev = jnp.where(low_half, deinterleave(v0, ev_sel),
                               deinterleave(v1, ev_sel))
                od = jnp.where(low_half, deinterleave(v0, od_sel),
                               deinterleave(v1, od_sel))
                pid_v[b, pl.ds(g * _L, _L)] = ev * 8 + od
            if _DO_GATHER:
                pltpu.make_async_copy(
                    tp_hbm.at[wid].at[pid_v.at[b]], pairs_f.at[b],
                    sem_g[b]).start()

        def wait_gather(b):
            if _DO_GATHER:
                pltpu.make_async_copy(dummy_hbm, pairs_f.at[b],
                                      sem_g[b]).wait()

        def fire_write(j, b):
            pb = pl.multiple_of(pair0 + j * _PAIRS, _PAIRS)
            if _DO_WRITE:
                pltpu.make_async_copy(
                    pairs_f.at[b], out_hbm.at[pl.ds(pb, _PAIRS)],
                    sem_w[b]).start()

        def drain_write(b):
            if _DO_WRITE:
                pltpu.make_async_copy(
                    pairs_f.at[b], out_hbm.at[pl.ds(0, _PAIRS)],
                    sem_w[b]).wait()

        ib = pl.multiple_of(idx0, _CHUNK)
        pltpu.sync_copy(idx_hbm.at[pl.ds(ib, per_w)], idx_all)

        def step(j, b, drain, fire):
            nxt = (b + 2) % _NBUF
            if drain:
                drain_write(nxt)
            if fire:
                fire_gather(j + 2, nxt)
            wait_gather(b)
            fire_write(j, b)

        fire_gather(0, 0)
        fire_gather(1, 1)
        for j in range(_NBUF):
            step(j, j, drain=(j >= 2), fire=True)

        def body(k_, carry):
            for b in range(_NBUF):
                step(k_ * _NBUF + b, b, drain=True, fire=True)
            return carry

        lax.fori_loop(1, n_chunks // _NBUF - 1, body, 0)

        for j in range(n_chunks - _NBUF, n_chunks):
            fire = j + 2 < n_chunks
            step(j, j % _NBUF, drain=fire, fire=fire)
        for b in range(_NBUF):
            drain_write(b)

    return k(tp, idx1d, dummy)


@functools.partial(jax.jit, static_argnames=("shape5",))
def _to_tminor(y2d, *, shape5):
    B, T, Hp, Wp, d = shape5
    y5 = y2d.reshape(B, T, Hp, Wp * d // 128, 128)

    def tbody(x_ref, z_ref):
        x2 = x_ref[0, :, 0].reshape(T, Wp * d)
        z_ref[0, 0] = jnp.transpose(x2).reshape(Wp, d, T)

    return pl.pallas_call(
        tbody,
        grid=(B, Hp),
        in_specs=[pl.BlockSpec((1, T, 1, Wp * d // 128, 128),
                               lambda b, h: (b, 0, h, 0, 0))],
        out_specs=pl.BlockSpec((1, 1, Wp, d, T), lambda b, h: (b, h, 0, 0, 0)),
        out_shape=jax.ShapeDtypeStruct((B, Hp, Wp, d, T), jnp.float32),
    )(y5)


def kernel(actions, table):
    B, T, Hp, Wp = actions.shape
    n = B * T * Hp * Wp
    v, d = table.shape
    idx1d = actions.reshape(n).astype(jnp.int32)
    tp = jnp.concatenate(
        [jnp.repeat(table, v, axis=0), jnp.tile(table, (v, 1))], axis=1)
    tp = jnp.broadcast_to(tp, (_NW,) + tp.shape)
    dummy = jnp.zeros((_PAIRS, 2 * d), jnp.float32)
    out2 = _emb_lookup(tp, idx1d, dummy, n=n, d=d)
    z = _to_tminor(out2, shape5=(B, T, Hp, Wp, d))
    return jnp.transpose(z, (0, 4, 1, 2, 3))


# speedup vs baseline: 1.5958x; 1.1398x over previous
import functools

import jax
import jax.numpy as jnp
from jax import lax
from jax.experimental import pallas as pl
from jax.experimental.pallas import tpu as pltpu
from jax.experimental.pallas import tpu_sc as plsc

_INFO = plsc.get_sparse_core_info()
_NC, _NS = _INFO.num_cores, _INFO.num_subcores
_NW = _NC * _NS
_L = 16

_NBUF = 4
_P = 4


@functools.partial(jax.jit, static_argnames=("n", "d", "ofs", "chunk"))
def _emb_lookup(tp, idx1d, dummy, *, n, d, ofs, chunk):
    per_w = n // _NW
    n_chunks = per_w // chunk
    pairs = chunk // 2
    assert pairs % _L == 0 and pairs <= 128
    assert (n_chunks - 2 * _NBUF) % _NBUF == 0 and n_chunks > 3 * _NBUF
    mesh = plsc.VectorSubcoreMesh(core_axis_name="c", subcore_axis_name="s")

    @functools.partial(
        pl.kernel,
        mesh=mesh,
        out_type=jax.ShapeDtypeStruct((n // 2, 2 * d), jnp.float32),
        scratch_types=[
            pltpu.VMEM((per_w,), jnp.int32),
            pltpu.VMEM((_NBUF, pairs), jnp.int32),
            pltpu.VMEM((_NBUF, pairs, 2 * d), jnp.float32),
            pltpu.SemaphoreType.DMA,
            pltpu.SemaphoreType.DMA,
            pltpu.SemaphoreType.DMA,
            pltpu.SemaphoreType.DMA,
            pltpu.SemaphoreType.DMA,
            pltpu.SemaphoreType.DMA,
            pltpu.SemaphoreType.DMA,
            pltpu.SemaphoreType.DMA,
        ],
    )
    def k(tp_hbm, idx_hbm, dummy_hbm, out_hbm, idx_all, pid_v, pairs_f,
          *sems):
        sem_g = sems[:_NBUF]
        sem_w = sems[_NBUF:]
        wid = lax.axis_index("s") * _NC + lax.axis_index("c")
        idx0 = ofs + wid * per_w
        pair0 = wid * (per_w // 2)
        lane = lax.iota(jnp.int32, _L)
        low_half = lane < 8
        ev_sel = jnp.arange(0, 2 * _L, 2, dtype=jnp.int32) % _L
        od_sel = ev_sel + 1

        def deinterleave(vv, sel):
            return vv.at[sel].get(mode="promise_in_bounds")

        def fire_gather(j, b):
            base = j * chunk
            for g in range(pairs // _L):
                v0 = idx_all[pl.ds(base + 2 * _L * g, _L)]
                v1 = idx_all[pl.ds(base + 2 * _L * g + _L, _L)]
                ev = jnp.where(low_half, deinterleave(v0, ev_sel),
                               deinterleave(v1, ev_sel))
                od = jnp.where(low_half, deinterleave(v0, od_sel),
                               deinterleave(v1, od_sel))
                pid_v[b, pl.ds(g * _L, _L)] = ev * 8 + od
            pltpu.make_async_copy(
                tp_hbm.at[wid].at[pid_v.at[b]], pairs_f.at[b],
                sem_g[b]).start()

        def wait_gather(b):
            pltpu.make_async_copy(dummy_hbm, pairs_f.at[b], sem_g[b]).wait()

        def fire_write(j, b):
            pb = pl.multiple_of(pair0 + j * pairs, pairs)
            pltpu.make_async_copy(
                pairs_f.at[b], out_hbm.at[pl.ds(pb, pairs)],
                sem_w[b]).start()

        def drain_write(b):
            pltpu.make_async_copy(
                pairs_f.at[b], out_hbm.at[pl.ds(0, pairs)], sem_w[b]).wait()

        ib = pl.multiple_of(idx0, chunk)
        pltpu.sync_copy(idx_hbm.at[pl.ds(ib, per_w)], idx_all)

        def step(j, b, drain, fire):
            nxt = (b + 2) % _NBUF
            if drain:
                drain_write(nxt)
            if fire:
                fire_gather(j + 2, nxt)
            wait_gather(b)
            fire_write(j, b)

        fire_gather(0, 0)
        fire_gather(1, 1)
        for j in range(_NBUF):
            step(j, j, drain=(j >= 2), fire=True)

        def body(k_, carry):
            for b in range(_NBUF):
                step(k_ * _NBUF + b, b, drain=True, fire=True)
            return carry

        lax.fori_loop(1, n_chunks // _NBUF - 1, body, 0)

        for j in range(n_chunks - _NBUF, n_chunks):
            fire = j + 2 < n_chunks
            step(j, j % _NBUF, drain=fire, fire=fire)
        for b in range(_NBUF):
            drain_write(b)

    return k(tp, idx1d, dummy)


def _to_tminor_piece(y2d, z_prev, *, p, shape5):
    B, T, Hp, Wp, d = shape5
    bp = B // _P
    y5 = y2d.reshape(bp, T, Hp, Wp * d // 128, 128)

    def tbody(x_ref, *rest):
        z_ref = rest[-1]
        x2 = x_ref[0, :, 0].reshape(T, Wp * d)
        z_ref[0, 0] = jnp.transpose(x2).reshape(Wp, d, T)

    in_specs = [pl.BlockSpec((1, T, 1, Wp * d // 128, 128),
                             lambda b, h: (b, 0, h, 0, 0))]
    args = [y5]
    kwargs = {}
    if z_prev is not None:
        in_specs.append(pl.BlockSpec(memory_space=pl.ANY))
        args.append(z_prev)
        kwargs["input_output_aliases"] = {1: 0}

    return pl.pallas_call(
        tbody,
        grid=(bp, Hp),
        in_specs=in_specs,
        out_specs=pl.BlockSpec((1, 1, Wp, d, T),
                               lambda b, h: (p * bp + b, h, 0, 0, 0)),
        out_shape=jax.ShapeDtypeStruct((B, Hp, Wp, d, T), jnp.float32),
        **kwargs,
    )(*args)


def kernel(actions, table):
    B, T, Hp, Wp = actions.shape
    n = B * T * Hp * Wp
    v, d = table.shape
    idx1d = actions.reshape(n).astype(jnp.int32)
    tp = jnp.concatenate(
        [jnp.repeat(table, v, axis=0), jnp.tile(table, (v, 1))], axis=1)
    tp = jnp.broadcast_to(tp, (_NW,) + tp.shape)
    n_p = n // _P
    chunk = 160
    dummy = jnp.zeros((chunk // 2, 2 * d), jnp.float32)
    ys = [_emb_lookup(tp, idx1d, dummy, n=n_p, d=d, ofs=p * n_p, chunk=chunk)
          for p in range(_P)]
    z = None
    for p in range(_P):
        z = _to_tminor_piece(ys[p], z, p=p, shape5=(B, T, Hp, Wp, d))
    return jnp.transpose(z, (0, 4, 1, 2, 3))


# speedup vs baseline: 1.8406x; 1.1534x over previous
import functools

import jax
import jax.numpy as jnp
from jax import lax
from jax.experimental import pallas as pl
from jax.experimental.pallas import tpu as pltpu
from jax.experimental.pallas import tpu_sc as plsc

_INFO = plsc.get_sparse_core_info()
_NC, _NS = _INFO.num_cores, _INFO.num_subcores
_NW = _NC * _NS
_L = 16

_NBUF = 4
_P = 4


@functools.partial(jax.jit, static_argnames=("n", "d", "ofs", "chunk"))
def _emb_lookup(tp, idx1d, dummy, *, n, d, ofs, chunk):
    per_w = n // _NW
    n_chunks = per_w // chunk
    pairs = chunk // 2
    assert pairs % _L == 0 and pairs <= 128
    assert (n_chunks - 2 * _NBUF) % _NBUF == 0 and n_chunks > 3 * _NBUF
    mesh = plsc.VectorSubcoreMesh(core_axis_name="c", subcore_axis_name="s")

    @functools.partial(
        pl.kernel,
        mesh=mesh,
        out_type=jax.ShapeDtypeStruct((n // 2, 2 * d), jnp.float32),
        scratch_types=[
            pltpu.VMEM((per_w,), jnp.int32),
            pltpu.VMEM((_NBUF, pairs), jnp.int32),
            pltpu.VMEM((_NBUF, pairs, 2 * d), jnp.float32),
            pltpu.VMEM_SHARED((_NS, 64, 2 * d), jnp.float32),
            pltpu.SemaphoreType.DMA,
            pltpu.SemaphoreType.DMA,
            pltpu.SemaphoreType.DMA,
            pltpu.SemaphoreType.DMA,
            pltpu.SemaphoreType.DMA,
            pltpu.SemaphoreType.DMA,
            pltpu.SemaphoreType.DMA,
            pltpu.SemaphoreType.DMA,
        ],
    )
    def k(tp_hbm, idx_hbm, dummy_hbm, out_hbm, idx_all, pid_v, pairs_f,
          tab_sp, *sems):
        sem_g = sems[:_NBUF]
        sem_w = sems[_NBUF:]
        sid = lax.axis_index("s")
        wid = lax.axis_index("s") * _NC + lax.axis_index("c")
        idx0 = ofs + wid * per_w
        pair0 = wid * (per_w // 2)
        lane = lax.iota(jnp.int32, _L)
        low_half = lane < 8
        ev_sel = jnp.arange(0, 2 * _L, 2, dtype=jnp.int32) % _L
        od_sel = ev_sel + 1

        def deinterleave(vv, sel):
            return vv.at[sel].get(mode="promise_in_bounds")

        def fire_gather(j, b):
            base = j * chunk
            for g in range(pairs // _L):
                v0 = idx_all[pl.ds(base + 2 * _L * g, _L)]
                v1 = idx_all[pl.ds(base + 2 * _L * g + _L, _L)]
                ev = jnp.where(low_half, deinterleave(v0, ev_sel),
                               deinterleave(v1, ev_sel))
                od = jnp.where(low_half, deinterleave(v0, od_sel),
                               deinterleave(v1, od_sel))
                pid_v[b, pl.ds(g * _L, _L)] = ev * 8 + od
            pltpu.make_async_copy(
                tab_sp.at[sid].at[pid_v.at[b]], pairs_f.at[b],
                sem_g[b]).start()

        def wait_gather(b):
            pltpu.make_async_copy(dummy_hbm, pairs_f.at[b], sem_g[b]).wait()

        def fire_write(j, b):
            pb = pl.multiple_of(pair0 + j * pairs, pairs)
            pltpu.make_async_copy(
                pairs_f.at[b], out_hbm.at[pl.ds(pb, pairs)],
                sem_w[b]).start()

        def drain_write(b):
            pltpu.make_async_copy(
                pairs_f.at[b], out_hbm.at[pl.ds(0, pairs)], sem_w[b]).wait()

        @pl.when(sid == 0)
        def _stage_table():
            for r in range(_NS):
                pltpu.sync_copy(tp_hbm.at[wid], tab_sp.at[r])

        plsc.subcore_barrier()

        ib = pl.multiple_of(idx0, chunk)
        pltpu.sync_copy(idx_hbm.at[pl.ds(ib, per_w)], idx_all)

        def step(j, b, drain, fire):
            nxt = (b + 2) % _NBUF
            if drain:
                drain_write(nxt)
            if fire:
                fire_gather(j + 2, nxt)
            wait_gather(b)
            fire_write(j, b)

        fire_gather(0, 0)
        fire_gather(1, 1)
        for j in range(_NBUF):
            step(j, j, drain=(j >= 2), fire=True)

        def body(k_, carry):
            for b in range(_NBUF):
                step(k_ * _NBUF + b, b, drain=True, fire=True)
            return carry

        lax.fori_loop(1, n_chunks // _NBUF - 1, body, 0)

        for j in range(n_chunks - _NBUF, n_chunks):
            fire = j + 2 < n_chunks
            step(j, j % _NBUF, drain=fire, fire=fire)
        for b in range(_NBUF):
            drain_write(b)

    return k(tp, idx1d, dummy)


def _to_tminor_piece(y2d, z_prev, *, p, shape5):
    B, T, Hp, Wp, d = shape5
    bp = B // _P
    y5 = y2d.reshape(bp, T, Hp, Wp * d // 128, 128)

    def tbody(x_ref, *rest):
        z_ref = rest[-1]
        x2 = x_ref[0, :, 0].reshape(T, Wp * d)
        z_ref[0, 0] = jnp.transpose(x2).reshape(Wp, d, T)

    in_specs = [pl.BlockSpec((1, T, 1, Wp * d // 128, 128),
                             lambda b, h: (b, 0, h, 0, 0))]
    args = [y5]
    kwargs = {}
    if z_prev is not None:
        in_specs.append(pl.BlockSpec(memory_space=pl.ANY))
        args.append(z_prev)
        kwargs["input_output_aliases"] = {1: 0}

    return pl.pallas_call(
        tbody,
        grid=(bp, Hp),
        in_specs=in_specs,
        out_specs=pl.BlockSpec((1, 1, Wp, d, T),
                               lambda b, h: (p * bp + b, h, 0, 0, 0)),
        out_shape=jax.ShapeDtypeStruct((B, Hp, Wp, d, T), jnp.float32),
        **kwargs,
    )(*args)


def kernel(actions, table):
    B, T, Hp, Wp = actions.shape
    n = B * T * Hp * Wp
    v, d = table.shape
    idx1d = actions.reshape(n).astype(jnp.int32)
    tp = jnp.concatenate(
        [jnp.repeat(table, v, axis=0), jnp.tile(table, (v, 1))], axis=1)
    tp = jnp.broadcast_to(tp, (_NW,) + tp.shape)
    n_p = n // _P
    chunk = 160
    dummy = jnp.zeros((chunk // 2, 2 * d), jnp.float32)
    ys = [_emb_lookup(tp, idx1d, dummy, n=n_p, d=d, ofs=p * n_p, chunk=chunk)
          for p in range(_P)]
    z = None
    for p in range(_P):
        z = _to_tminor_piece(ys[p], z, p=p, shape5=(B, T, Hp, Wp, d))
    return jnp.transpose(z, (0, 4, 1, 2, 3))
